# Initial kernel scaffold; baseline (speedup 1.0000x reference)
import jax
import jax.numpy as jnp
from jax.experimental import pallas as pl


def kernel(x, feature, W, num_pool):
    raise NotImplementedError("write your pallas kernel here")


# speedup vs baseline: 14.7726x; 14.7726x over previous
import functools

import jax
import jax.numpy as jnp
from jax import lax
from jax.experimental import pallas as pl
from jax.experimental.pallas import tpu as pltpu
from jax.experimental.pallas import tpu_sc as plsc

_B = 8
_N = 4096
_C = 128
_K = 32
_S = 1024
_R2 = jnp.float32(0.04)
_NCHUNK = _N // 16
_NW = 32
_QPW = (_B * _S) // _NW


def _fps_tc_kernel(x_ref, far0_ref, qx_ref, qy_ref, qz_ref):
    X = x_ref[:, 0, :]
    Y = x_ref[:, 1, :]
    Z = x_ref[:, 2, :]
    lane_n = lax.broadcasted_iota(jnp.int32, (_B, _N), 1)
    lane_s = lax.broadcasted_iota(jnp.int32, (_B, _S), 1)

    def body(i, carry):
        dist, far, Qx, Qy, Qz = carry
        oh = lane_n == far
        cx = jnp.sum(jnp.where(oh, X, 0.0), axis=1, keepdims=True)
        cy = jnp.sum(jnp.where(oh, Y, 0.0), axis=1, keepdims=True)
        cz = jnp.sum(jnp.where(oh, Z, 0.0), axis=1, keepdims=True)
        ohq = lane_s == i
        Qx = jnp.where(ohq, cx, Qx)
        Qy = jnp.where(ohq, cy, Qy)
        Qz = jnp.where(ohq, cz, Qz)
        dx = X - cx
        dy = Y - cy
        dz = Z - cz
        d = (dx * dx + dy * dy) + dz * dz
        dist = jnp.minimum(dist, d)
        m = jnp.max(dist, axis=1, keepdims=True)
        far = jnp.min(jnp.where(dist == m, lane_n, _N), axis=1, keepdims=True)
        return dist, far, Qx, Qy, Qz

    dist0 = jnp.full((_B, _N), 1e10, jnp.float32)
    far0 = far0_ref[:, :]
    Q0 = jnp.zeros((_B, _S), jnp.float32)
    _, _, Qx, Qy, Qz = lax.fori_loop(0, _S, body, (dist0, far0, Q0, Q0, Q0))
    qx_ref[:, :] = Qx
    qy_ref[:, :] = Qy
    qz_ref[:, :] = Qz


def _fps(x, far0):
    return pl.pallas_call(
        _fps_tc_kernel,
        out_shape=[jax.ShapeDtypeStruct((_B, _S), jnp.float32)] * 3,
    )(x, far0)


def _mm_kernel(f_ref, w_ref, o_ref):
    o_ref[...] = jnp.dot(f_ref[0], w_ref[...],
                         preferred_element_type=jnp.float32)[None]


def _conv_all_points(feature_t, w_t):
    nt = 8
    blk = _N // nt
    return pl.pallas_call(
        _mm_kernel,
        grid=(_B, nt),
        in_specs=[
            pl.BlockSpec((1, blk, _C), lambda b, n: (b, n, 0)),
            pl.BlockSpec((_C, _C), lambda b, n: (0, 0)),
        ],
        out_specs=pl.BlockSpec((1, blk, _C), lambda b, n: (b, n, 0)),
        out_shape=jax.ShapeDtypeStruct((_B, _N, _C), jnp.float32),
    )(feature_t, w_t)


def _rbf16(v):
    u = plsc.bitcast(v, jnp.uint32)
    r = (u >> jnp.uint32(16)) & jnp.uint32(1)
    u = (u + jnp.uint32(0x7FFF) + r) & jnp.uint32(0xFFFF0000)
    return plsc.bitcast(u, jnp.float32)


def _sc_kernel(x_hbm, qx_hbm, qy_hbm, qz_hbm, wf_hbm,
               xout_hbm, fout_hbm,
               px, py, pz, sqx, rx, ry, rz, qx, qy, qz, buf, rows, fl, xl,
               sem):
    w = lax.axis_index("s") * 2 + lax.axis_index("c")
    b = w // 4
    s0 = (w % 4) * _QPW

    pltpu.sync_copy(x_hbm.at[pl.ds((b * 3 + 0) * _N, _N)], px)
    pltpu.sync_copy(x_hbm.at[pl.ds((b * 3 + 1) * _N, _N)], py)
    pltpu.sync_copy(x_hbm.at[pl.ds((b * 3 + 2) * _N, _N)], pz)
    pltpu.sync_copy(qx_hbm.at[pl.ds(b * _S + s0, _QPW)], qx)
    pltpu.sync_copy(qy_hbm.at[pl.ds(b * _S + s0, _QPW)], qy)
    pltpu.sync_copy(qz_hbm.at[pl.ds(b * _S + s0, _QPW)], qz)

    def sq_body(j, _):
        o = j * 16
        pxv = px[pl.ds(o, 16)]
        pyv = py[pl.ds(o, 16)]
        pzv = pz[pl.ds(o, 16)]
        sqx[pl.ds(o, 16)] = (pxv * pxv + pyv * pyv) + pzv * pzv
        rx[pl.ds(o, 16)] = _rbf16(pxv)
        ry[pl.ds(o, 16)] = _rbf16(pyv)
        rz[pl.ds(o, 16)] = _rbf16(pzv)
        return 0

    lax.fori_loop(0, _NCHUNK, sq_body, 0)

    lanes = lax.iota(jnp.int32, 16)
    zeros16 = jnp.zeros((16,), jnp.int32)
    lane0 = lanes == 0

    def q_body(s, _):
        sfull = jnp.full((16,), s, jnp.int32)
        qxv = plsc.load_gather(qx, [sfull])
        qyv = plsc.load_gather(qy, [sfull])
        qzv = plsc.load_gather(qz, [sfull])
        sqqv = (qxv * qxv + qyv * qyv) + qzv * qzv
        qxb = _rbf16(qxv)
        qyb = _rbf16(qyv)
        qzb = _rbf16(qzv)

        def cond(st):
            off, jc = st
            return (off < _K) & (jc < _NCHUNK)

        def wbody(st):
            off, jc = st
            o = jc * 16
            pxv = rx[pl.ds(o, 16)]
            pyv = ry[pl.ds(o, 16)]
            pzv = rz[pl.ds(o, 16)]
            sxv = sqx[pl.ds(o, 16)]
            dot = (qxb * pxv + qyb * pyv) + qzb * pzv
            d = (sqqv + sxv) - 2.0 * dot
            msk = d <= _R2
            ranks = plsc.cumsum(msk.astype(jnp.int32))
            sel = msk & (ranks <= (_K - off))
            idxv = jnp.full((16,), b * _N + o, jnp.int32) + lanes
            plsc.store_compressed(buf.at[pl.ds(s * 64 + off, 16)], idxv,
                                  mask=sel)
            cnt = jnp.sum(sel.astype(jnp.int32))
            return off + cnt, jc + 1

        off, _ = lax.while_loop(cond, wbody, (jnp.int32(0), jnp.int32(0)))

        fv = plsc.load_gather(buf, [jnp.full((16,), s * 64, jnp.int32)])
        buf[pl.ds(s * 64 + off, 16)] = fv
        buf[pl.ds(s * 64 + off + 16, 16)] = fv

        i1 = buf[pl.ds(s * 64, 16)] - (b * _N)
        i2 = buf[pl.ds(s * 64 + 16, 16)] - (b * _N)
        xm = jnp.max(jnp.maximum(plsc.load_gather(px, [i1]),
                                 plsc.load_gather(px, [i2])))
        ym = jnp.max(jnp.maximum(plsc.load_gather(py, [i1]),
                                 plsc.load_gather(py, [i2])))
        zm = jnp.max(jnp.maximum(plsc.load_gather(pz, [i1]),
                                 plsc.load_gather(pz, [i2])))
        plsc.store_scatter(xl, [sfull], jnp.full((16,), xm, jnp.float32),
                           mask=lane0)
        plsc.store_scatter(xl, [sfull + _QPW],
                           jnp.full((16,), ym, jnp.float32), mask=lane0)
        plsc.store_scatter(xl, [sfull + 2 * _QPW],
                           jnp.full((16,), zm, jnp.float32), mask=lane0)

        pltpu.async_copy(wf_hbm.at[buf.at[pl.ds(s * 64, _K)]], rows,
                         sem).wait()

        acc = [rows[0, pl.ds(ci * 16, 16)] for ci in range(8)]
        for r in range(1, _K):
            for ci in range(8):
                acc[ci] = jnp.maximum(acc[ci], rows[r, pl.ds(ci * 16, 16)])
        for ci in range(8):
            fl[pl.ds(s * _C + ci * 16, 16)] = acc[ci]
        return 0

    lax.fori_loop(0, _QPW, q_body, 0)

    pltpu.sync_copy(fl, fout_hbm.at[pl.ds(w * _QPW * _C, _QPW * _C)])
    pltpu.sync_copy(xl.at[pl.ds(0, _QPW)],
                    xout_hbm.at[pl.ds((b * 3 + 0) * _S + s0, _QPW)])
    pltpu.sync_copy(xl.at[pl.ds(_QPW, _QPW)],
                    xout_hbm.at[pl.ds((b * 3 + 1) * _S + s0, _QPW)])
    pltpu.sync_copy(xl.at[pl.ds(2 * _QPW, _QPW)],
                    xout_hbm.at[pl.ds((b * 3 + 2) * _S + s0, _QPW)])


def _sc_call(x, qx, qy, qz, wf_rows):
    mesh = plsc.VectorSubcoreMesh(core_axis_name="c", subcore_axis_name="s")
    f = functools.partial(
        pl.kernel,
        mesh=mesh,
        compiler_params=pltpu.CompilerParams(needs_layout_passes=False),
        out_type=[
            jax.ShapeDtypeStruct((_B * 3 * _S,), jnp.float32),
            jax.ShapeDtypeStruct((_B * _S * _C,), jnp.float32),
        ],
        scratch_types=[
            pltpu.VMEM((_N,), jnp.float32),
            pltpu.VMEM((_N,), jnp.float32),
            pltpu.VMEM((_N,), jnp.float32),
            pltpu.VMEM((_N,), jnp.float32),
            pltpu.VMEM((_N,), jnp.float32),
            pltpu.VMEM((_N,), jnp.float32),
            pltpu.VMEM((_N,), jnp.float32),
            pltpu.VMEM((_QPW,), jnp.float32),
            pltpu.VMEM((_QPW,), jnp.float32),
            pltpu.VMEM((_QPW,), jnp.float32),
            pltpu.VMEM((_QPW * 64,), jnp.int32),
            pltpu.VMEM((_K, _C), jnp.float32),
            pltpu.VMEM((_QPW * _C,), jnp.float32),
            pltpu.VMEM((3 * _QPW,), jnp.float32),
            pltpu.SemaphoreType.DMA,
        ],
    )(_sc_kernel)
    return f(x, qx, qy, qz, wf_rows)


def kernel(x, feature, W, num_pool):
    residual = jnp.asarray(num_pool, jnp.int32) - _S
    far0 = jnp.zeros((_B, 1), jnp.int32) + residual
    qx, qy, qz = _fps(x, far0)
    wf = _conv_all_points(
        jnp.transpose(feature, (0, 2, 1)), jnp.transpose(W))
    wf_rows = wf.reshape(_B * _N, _C)
    xout, fout = _sc_call(x.reshape(-1), qx.reshape(-1), qy.reshape(-1),
                          qz.reshape(-1), wf_rows)
    x_out = xout.reshape(_B, 3, _S)
    f_out = jnp.transpose(fout.reshape(_B, _S, _C), (0, 2, 1))
    return (x_out, f_out)


# speedup vs baseline: 17.2262x; 1.1661x over previous
import functools

import jax
import jax.numpy as jnp
from jax import lax
from jax.experimental import pallas as pl
from jax.experimental.pallas import tpu as pltpu
from jax.experimental.pallas import tpu_sc as plsc

_B = 8
_N = 4096
_C = 128
_K = 32
_S = 1024
_R2 = jnp.float32(0.04)
_NCHUNK = _N // 16
_NW = 32
_QPW = (_B * _S) // _NW
_RING = 4


def _fps_tc_kernel(x_ref, far0_ref, qx_ref, qy_ref, qz_ref):
    X = x_ref[:, 0, :]
    Y = x_ref[:, 1, :]
    Z = x_ref[:, 2, :]
    lane_n = lax.broadcasted_iota(jnp.int32, (_B, _N), 1)
    lane_s = lax.broadcasted_iota(jnp.int32, (_B, _S), 1)

    def body(i, carry):
        dist, far, Qx, Qy, Qz = carry
        oh = lane_n == far
        cx = jnp.sum(jnp.where(oh, X, 0.0), axis=1, keepdims=True)
        cy = jnp.sum(jnp.where(oh, Y, 0.0), axis=1, keepdims=True)
        cz = jnp.sum(jnp.where(oh, Z, 0.0), axis=1, keepdims=True)
        ohq = lane_s == i
        Qx = jnp.where(ohq, cx, Qx)
        Qy = jnp.where(ohq, cy, Qy)
        Qz = jnp.where(ohq, cz, Qz)
        dx = X - cx
        dy = Y - cy
        dz = Z - cz
        d = (dx * dx + dy * dy) + dz * dz
        dist = jnp.minimum(dist, d)
        m = jnp.max(dist, axis=1, keepdims=True)
        far = jnp.min(jnp.where(dist == m, lane_n, _N), axis=1, keepdims=True)
        return dist, far, Qx, Qy, Qz

    dist0 = jnp.full((_B, _N), 1e10, jnp.float32)
    far0 = far0_ref[:, :]
    Q0 = jnp.zeros((_B, _S), jnp.float32)
    _, _, Qx, Qy, Qz = lax.fori_loop(0, _S, body, (dist0, far0, Q0, Q0, Q0))
    qx_ref[:, :] = Qx
    qy_ref[:, :] = Qy
    qz_ref[:, :] = Qz


def _fps(x, far0):
    return pl.pallas_call(
        _fps_tc_kernel,
        out_shape=[jax.ShapeDtypeStruct((_B, _S), jnp.float32)] * 3,
    )(x, far0)


def _mm_kernel(f_ref, w_ref, o_ref):
    o_ref[...] = jnp.dot(f_ref[0], w_ref[...],
                         preferred_element_type=jnp.float32)[None]


def _conv_all_points(feature_t, w_t):
    nt = 8
    blk = _N // nt
    return pl.pallas_call(
        _mm_kernel,
        grid=(_B, nt),
        in_specs=[
            pl.BlockSpec((1, blk, _C), lambda b, n: (b, n, 0)),
            pl.BlockSpec((_C, _C), lambda b, n: (0, 0)),
        ],
        out_specs=pl.BlockSpec((1, blk, _C), lambda b, n: (b, n, 0)),
        out_shape=jax.ShapeDtypeStruct((_B, _N, _C), jnp.float32),
    )(feature_t, w_t)


def _rbf16(v):
    u = plsc.bitcast(v, jnp.uint32)
    r = (u >> jnp.uint32(16)) & jnp.uint32(1)
    u = (u + jnp.uint32(0x7FFF) + r) & jnp.uint32(0xFFFF0000)
    return plsc.bitcast(u, jnp.float32)


def _sc_kernel(x_hbm, qx_hbm, qy_hbm, qz_hbm, wf_hbm,
               xout_hbm, fout_hbm,
               px, py, pz, sqx, rx, ry, rz, qx, qy, qz, buf, rows, fl, xl,
               sems):
    w = lax.axis_index("s") * 2 + lax.axis_index("c")
    b = w // 4
    s0 = (w % 4) * _QPW

    pltpu.sync_copy(x_hbm.at[pl.ds((b * 3 + 0) * _N, _N)], px)
    pltpu.sync_copy(x_hbm.at[pl.ds((b * 3 + 1) * _N, _N)], py)
    pltpu.sync_copy(x_hbm.at[pl.ds((b * 3 + 2) * _N, _N)], pz)
    pltpu.sync_copy(qx_hbm.at[pl.ds(b * _S + s0, _QPW)], qx)
    pltpu.sync_copy(qy_hbm.at[pl.ds(b * _S + s0, _QPW)], qy)
    pltpu.sync_copy(qz_hbm.at[pl.ds(b * _S + s0, _QPW)], qz)

    def sq_body(j, _):
        o = j * 16
        pxv = px[pl.ds(o, 16)]
        pyv = py[pl.ds(o, 16)]
        pzv = pz[pl.ds(o, 16)]
        sqx[pl.ds(o, 16)] = (pxv * pxv + pyv * pyv) + pzv * pzv
        rx[pl.ds(o, 16)] = _rbf16(pxv)
        ry[pl.ds(o, 16)] = _rbf16(pyv)
        rz[pl.ds(o, 16)] = _rbf16(pzv)
        return 0

    lax.fori_loop(0, _NCHUNK, sq_body, 0)

    lanes = lax.iota(jnp.int32, 16)
    lane0 = lanes == 0

    def q_body(s, _):
        sfull = jnp.full((16,), s, jnp.int32)
        qxv = plsc.load_gather(qx, [sfull])
        qyv = plsc.load_gather(qy, [sfull])
        qzv = plsc.load_gather(qz, [sfull])
        sqqv = (qxv * qxv + qyv * qyv) + qzv * qzv
        q2x = _rbf16(qxv) * 2.0
        q2y = _rbf16(qyv) * 2.0
        q2z = _rbf16(qzv) * 2.0

        def cond(st):
            off, jc = st
            return (off < _K) & (jc < _NCHUNK)

        def wbody(st):
            off, jc = st
            o = jc * 16
            pxv = rx[pl.ds(o, 16)]
            pyv = ry[pl.ds(o, 16)]
            pzv = rz[pl.ds(o, 16)]
            sxv = sqx[pl.ds(o, 16)]
            dot2 = (q2x * pxv + q2y * pyv) + q2z * pzv
            d = (sqqv + sxv) - dot2
            msk = d <= _R2
            cnt = plsc.all_reduce_population_count(msk)[0]
            idxv = jnp.full((16,), b * _N + o, jnp.int32) + lanes

            def fits(_):
                plsc.store_compressed(buf.at[pl.ds(s * 64 + off, 16)], idxv,
                                      mask=msk)
                return off + cnt

            def capped(_):
                ranks = plsc.cumsum(msk.astype(jnp.int32))
                sel = msk & (ranks <= (_K - off))
                plsc.store_compressed(buf.at[pl.ds(s * 64 + off, 16)], idxv,
                                      mask=sel)
                return jnp.int32(_K)

            off = lax.cond(off + cnt <= _K, fits, capped, 0)
            return off, jc + 1

        off, _ = lax.while_loop(cond, wbody, (jnp.int32(0), jnp.int32(0)))

        fv = plsc.load_gather(buf, [jnp.full((16,), s * 64, jnp.int32)])
        buf[pl.ds(s * 64 + off, 16)] = fv
        buf[pl.ds(s * 64 + off + 16, 16)] = fv

        i1 = buf[pl.ds(s * 64, 16)] - (b * _N)
        i2 = buf[pl.ds(s * 64 + 16, 16)] - (b * _N)
        xm = jnp.max(jnp.maximum(plsc.load_gather(px, [i1]),
                                 plsc.load_gather(px, [i2])))
        ym = jnp.max(jnp.maximum(plsc.load_gather(py, [i1]),
                                 plsc.load_gather(py, [i2])))
        zm = jnp.max(jnp.maximum(plsc.load_gather(pz, [i1]),
                                 plsc.load_gather(pz, [i2])))
        plsc.store_scatter(xl, [sfull], jnp.full((16,), xm, jnp.float32),
                           mask=lane0)
        plsc.store_scatter(xl, [sfull + _QPW],
                           jnp.full((16,), ym, jnp.float32), mask=lane0)
        plsc.store_scatter(xl, [sfull + 2 * _QPW],
                           jnp.full((16,), zm, jnp.float32), mask=lane0)
        return 0

    lax.fori_loop(0, _QPW, q_body, 0)

    def gather_to(s, t):
        return pltpu.async_copy(wf_hbm.at[buf.at[pl.ds(s * 64, _K)]],
                                rows.at[t], sems[t])

    def reduce_slot(s, t):
        acc = [rows[t, 0, pl.ds(ci * 16, 16)] for ci in range(8)]
        for r in range(1, _K):
            for ci in range(8):
                acc[ci] = jnp.maximum(acc[ci],
                                      rows[t, r, pl.ds(ci * 16, 16)])
        for ci in range(8):
            fl[pl.ds(s * _C + ci * 16, 16)] = acc[ci]

    def g_body(g, _):
        s = g * _RING
        cps = [gather_to(s + t, t) for t in range(_RING)]
        for t in range(_RING):
            cps[t].wait()
            reduce_slot(s + t, t)
        return 0

    lax.fori_loop(0, _QPW // _RING, g_body, 0)

    pltpu.sync_copy(fl, fout_hbm.at[pl.ds(w * _QPW * _C, _QPW * _C)])
    pltpu.sync_copy(xl.at[pl.ds(0, _QPW)],
                    xout_hbm.at[pl.ds((b * 3 + 0) * _S + s0, _QPW)])
    pltpu.sync_copy(xl.at[pl.ds(_QPW, _QPW)],
                    xout_hbm.at[pl.ds((b * 3 + 1) * _S + s0, _QPW)])
    pltpu.sync_copy(xl.at[pl.ds(2 * _QPW, _QPW)],
                    xout_hbm.at[pl.ds((b * 3 + 2) * _S + s0, _QPW)])


def _sc_call(x, qx, qy, qz, wf_rows):
    mesh = plsc.VectorSubcoreMesh(core_axis_name="c", subcore_axis_name="s")
    f = functools.partial(
        pl.kernel,
        mesh=mesh,
        compiler_params=pltpu.CompilerParams(needs_layout_passes=False),
        out_type=[
            jax.ShapeDtypeStruct((_B * 3 * _S,), jnp.float32),
            jax.ShapeDtypeStruct((_B * _S * _C,), jnp.float32),
        ],
        scratch_types=[
            pltpu.VMEM((_N,), jnp.float32),
            pltpu.VMEM((_N,), jnp.float32),
            pltpu.VMEM((_N,), jnp.float32),
            pltpu.VMEM((_N,), jnp.float32),
            pltpu.VMEM((_N,), jnp.float32),
            pltpu.VMEM((_N,), jnp.float32),
            pltpu.VMEM((_N,), jnp.float32),
            pltpu.VMEM((_QPW,), jnp.float32),
            pltpu.VMEM((_QPW,), jnp.float32),
            pltpu.VMEM((_QPW,), jnp.float32),
            pltpu.VMEM((_QPW * 64,), jnp.int32),
            pltpu.VMEM((_RING, _K, _C), jnp.float32),
            pltpu.VMEM((_QPW * _C,), jnp.float32),
            pltpu.VMEM((3 * _QPW,), jnp.float32),
            [pltpu.SemaphoreType.DMA] * _RING,
        ],
    )(_sc_kernel)
    return f(x, qx, qy, qz, wf_rows)


def kernel(x, feature, W, num_pool):
    residual = jnp.asarray(num_pool, jnp.int32) - _S
    far0 = jnp.zeros((_B, 1), jnp.int32) + residual
    qx, qy, qz = _fps(x, far0)
    wf = _conv_all_points(
        jnp.transpose(feature, (0, 2, 1)), jnp.transpose(W))
    wf_rows = wf.reshape(_B * _N, _C)
    xout, fout = _sc_call(x.reshape(-1), qx.reshape(-1), qy.reshape(-1),
                          qz.reshape(-1), wf_rows)
    x_out = xout.reshape(_B, 3, _S)
    f_out = jnp.transpose(fout.reshape(_B, _S, _C), (0, 2, 1))
    return (x_out, f_out)
